# baseline (speedup 1.0000x reference)
import jax
import jax.numpy as jnp
from jax import lax
from jax.experimental import pallas as pl
from jax.experimental.pallas import tpu as pltpu
from jax.experimental.pallas import tpu_sc as plsc

B = 16384
HALF = B // 2
DIM = 64
NENT = 1000000
MARGIN = 1.0
NC = 2
NS = 16
NW = NC * NS
CHUNK = HALF // NW
L = 16
SUB = 128
NPHASE = 2 * CHUNK // SUB
NG = SUB // L
EWORDS = 2 * SUB * DIM


def _rsqrt_newton(x):
    bits = lax.bitcast_convert_type(x, jnp.int32)
    y = lax.bitcast_convert_type(
        jnp.int32(0x5F3759DF) - lax.shift_right_logical(bits, 1), jnp.float32)
    half_x = 0.5 * x
    for _ in range(3):
        y = y * (1.5 - half_x * y * y)
    return y


def _transe_body(bh_hbm, bt_hbm, br_hbm, ent_hbm, rel_hbm,
                 pos_out, neg_out, part_out,
                 ihb, itb, irb, ridx, ei0, ei1, eb0, eb1, rb0, rb1,
                 psb, nsb, pvec, gsem, rsem):
    wid = lax.axis_index("s") * NC + lax.axis_index("c")
    pbase = wid * CHUNK
    nbase = HALF + wid * CHUNK

    pltpu.sync_copy(bh_hbm.at[pl.ds(pbase, CHUNK)], ihb.at[pl.ds(0, CHUNK)])
    pltpu.sync_copy(bh_hbm.at[pl.ds(nbase, CHUNK)], ihb.at[pl.ds(CHUNK, CHUNK)])
    pltpu.sync_copy(bt_hbm.at[pl.ds(pbase, CHUNK)], itb.at[pl.ds(0, CHUNK)])
    pltpu.sync_copy(bt_hbm.at[pl.ds(nbase, CHUNK)], itb.at[pl.ds(CHUNK, CHUNK)])
    pltpu.sync_copy(br_hbm.at[pl.ds(pbase, CHUNK)], irb.at[pl.ds(0, CHUNK)])
    pltpu.sync_copy(br_hbm.at[pl.ds(nbase, CHUNK)], irb.at[pl.ds(CHUNK, CHUNK)])

    def rbuild_body(g, carry):
        rv = irb[pl.ds(g * L, L)]
        ridx[pl.ds(g * L, L)] = lax.shift_right_logical(rv, 1)
        return carry

    lax.fori_loop(0, 2 * CHUNK // L, rbuild_body, 0)

    def build_phase(k, ei):
        def g_body(g, carry):
            hv = ihb[pl.ds(k * SUB + g * L, L)]
            tv = itb[pl.ds(k * SUB + g * L, L)]

            def j_body(j, carry2):
                ah, at = carry2
                off = (g * DIM + j) * L
                ei[pl.ds(off, L)] = ah
                ei[pl.ds(SUB * DIM + off, L)] = at
                return (ah + NENT, at + NENT)

            lax.fori_loop(0, DIM, j_body, (hv, tv), unroll=8)
            return carry

        lax.fori_loop(0, NG, g_body, 0)

    def fire_phase(k, ei, eb, rb):
        cp = pltpu.async_copy(ent_hbm.at[ei], eb, gsem)
        cr = pltpu.async_copy(rel_hbm.at[ridx.at[pl.ds(k * SUB, SUB)]], rb, rsem)
        return (cp, cr)

    eibufs = (ei0, ei1)
    ebufs = (eb0, eb1)
    rbufs = (rb0, rb1)

    build_phase(0, ei0)
    pending = [fire_phase(0, ei0, eb0, rb0)]
    build_phase(1, ei1)
    pending.append(fire_phase(1, ei1, eb1, rb1))

    def compute_phase(k, eb, rb):
        sout = psb if k < NPHASE // 2 else nsb
        sbase = (k % (NPHASE // 2)) * SUB

        def g_body(g, carry):
            rrows = g * L + lax.iota(jnp.int32, L)
            rbase = (irb[pl.ds(k * SUB + g * L, L)] & 1) * DIM

            def j_body(j, acc):
                h2, t2, r2, hr, tr, ht = acc
                off = (g * DIM + j) * L
                hv = eb[pl.ds(off, L)]
                tv = eb[pl.ds(SUB * DIM + off, L)]
                rv = plsc.load_gather(
                    rb, [rrows, rbase + jnp.full((L,), j, dtype=jnp.int32)])
                return (h2 + hv * hv, t2 + tv * tv, r2 + rv * rv,
                        hr + hv * rv, tr + tv * rv, ht + hv * tv)

            zeros = jnp.zeros((L,), jnp.float32)
            h2, t2, r2, hr, tr, ht = lax.fori_loop(
                0, DIM, j_body, (zeros,) * 6, unroll=8)

            sh = jnp.minimum(1.0, _rsqrt_newton(h2))
            st = jnp.minimum(1.0, _rsqrt_newton(t2))
            dsq = (sh * sh * h2 + r2 + st * st * t2
                   + 2.0 * sh * hr - 2.0 * st * tr - 2.0 * (sh * st) * ht)
            dsq = jnp.maximum(dsq, 0.0)
            sout[pl.ds(sbase + g * L, L)] = dsq * _rsqrt_newton(dsq)
            return carry

        lax.fori_loop(0, NG, g_body, 0)

    for k in range(NPHASE):
        cp, cr = pending[k]
        cp.wait()
        cr.wait()
        compute_phase(k, ebufs[k % 2], rbufs[k % 2])
        nxt = k + 2
        if nxt < NPHASE:
            build_phase(nxt, eibufs[nxt % 2])
            pending.append(fire_phase(
                nxt, eibufs[nxt % 2], ebufs[nxt % 2], rbufs[nxt % 2]))

    def l_body(g, acc):
        p = psb[pl.ds(g * L, L)]
        n = nsb[pl.ds(g * L, L)]
        return acc + jnp.maximum(0.0, p - n + MARGIN)

    lacc = lax.fori_loop(0, CHUNK // L, l_body, jnp.zeros((L,), jnp.float32))
    pvec[...] = jnp.full((L,), jnp.sum(lacc), jnp.float32)

    pltpu.sync_copy(psb, pos_out.at[pl.ds(pbase, CHUNK)])
    pltpu.sync_copy(nsb, neg_out.at[pl.ds(pbase, CHUNK)])
    pltpu.sync_copy(pvec, part_out.at[wid])


@jax.jit
def _transe_sc(bh, bt, br, ent_flat, rel2):
    mesh = plsc.VectorSubcoreMesh(
        core_axis_name="c", subcore_axis_name="s",
        num_cores=NC, num_subcores=NS)
    f = pl.kernel(
        _transe_body,
        out_type=(
            jax.ShapeDtypeStruct((HALF,), jnp.float32),
            jax.ShapeDtypeStruct((HALF,), jnp.float32),
            jax.ShapeDtypeStruct((NW, L), jnp.float32),
        ),
        mesh=mesh,
        compiler_params=pltpu.CompilerParams(
            needs_layout_passes=False, use_tc_tiling_on_sc=False),
        scratch_types=(
            [pltpu.VMEM((2 * CHUNK,), jnp.int32) for _ in range(4)]
            + [pltpu.VMEM((EWORDS,), jnp.int32) for _ in range(2)]
            + [pltpu.VMEM((EWORDS,), jnp.float32) for _ in range(2)]
            + [pltpu.VMEM((SUB, 2 * DIM), jnp.float32) for _ in range(2)]
            + [pltpu.VMEM((CHUNK,), jnp.float32) for _ in range(2)]
            + [pltpu.VMEM((L,), jnp.float32),
               pltpu.SemaphoreType.DMA, pltpu.SemaphoreType.DMA]
        ),
    )
    return f(bh, bt, br, ent_flat, rel2)


def kernel(batch_h, batch_t, batch_r, batch_y, ent_emb, rel_emb):
    del batch_y
    bh = batch_h.astype(jnp.int32)
    bt = batch_t.astype(jnp.int32)
    br = batch_r.astype(jnp.int32)
    ent_flat = ent_emb.T.reshape(-1)
    rel2 = rel_emb.reshape(rel_emb.shape[0] // 2, 2 * DIM)
    pos_score, neg_score, partials = _transe_sc(bh, bt, br, ent_flat, rel2)
    loss = jnp.sum(partials[:, 0])
    return (loss, pos_score, neg_score)


# speedup vs baseline: 7.6455x; 7.6455x over previous
import jax
import jax.numpy as jnp
from jax import lax
from jax.experimental import pallas as pl
from jax.experimental.pallas import tpu as pltpu
from jax.experimental.pallas import tpu_sc as plsc

B = 16384
HALF = B // 2
DIM = 64
NENT = 1000000
MARGIN = 1.0
NC = 2
NS = 16
NW = NC * NS
CHUNK = HALF // NW
L = 16
SUB = 128
NPHASE = 2 * CHUNK // SUB
NG = SUB // L


def _rsqrt_newton(x):
    bits = lax.bitcast_convert_type(x, jnp.int32)
    y = lax.bitcast_convert_type(
        jnp.int32(0x5F3759DF) - lax.shift_right_logical(bits, 1), jnp.float32)
    half_x = 0.5 * x
    for _ in range(3):
        y = y * (1.5 - half_x * y * y)
    return y


def _transe_body(bh_hbm, bt_hbm, br_hbm, ent_hbm, rel_hbm,
                 pos_out, neg_out, part_out,
                 ihb, itb, irb, gidx, ridx, eb0, eb1, rb0, rb1,
                 psb, nsb, pvec, gsem, rsem):
    wid = lax.axis_index("s") * NC + lax.axis_index("c")
    pbase = wid * CHUNK
    nbase = HALF + wid * CHUNK

    pltpu.sync_copy(bh_hbm.at[pl.ds(pbase, CHUNK)], ihb.at[pl.ds(0, CHUNK)])
    pltpu.sync_copy(bh_hbm.at[pl.ds(nbase, CHUNK)], ihb.at[pl.ds(CHUNK, CHUNK)])
    pltpu.sync_copy(bt_hbm.at[pl.ds(pbase, CHUNK)], itb.at[pl.ds(0, CHUNK)])
    pltpu.sync_copy(bt_hbm.at[pl.ds(nbase, CHUNK)], itb.at[pl.ds(CHUNK, CHUNK)])
    pltpu.sync_copy(br_hbm.at[pl.ds(pbase, CHUNK)], irb.at[pl.ds(0, CHUNK)])
    pltpu.sync_copy(br_hbm.at[pl.ds(nbase, CHUNK)], irb.at[pl.ds(CHUNK, CHUNK)])

    def build_body(g, carry):
        hv = ihb[pl.ds(g * L, L)]
        tv = itb[pl.ds(g * L, L)]
        k = g // NG
        off = k * 2 * SUB + (g % NG) * L
        gidx[pl.ds(off, L)] = lax.shift_right_logical(hv, 1)
        gidx[pl.ds(off + SUB, L)] = lax.shift_right_logical(tv, 1)
        ridx[pl.ds(g * L, L)] = lax.shift_right_logical(irb[pl.ds(g * L, L)], 1)
        return carry

    lax.fori_loop(0, 2 * CHUNK // L, build_body, 0)

    def fire_phase(k, eb, rb):
        cp = pltpu.async_copy(
            ent_hbm.at[gidx.at[pl.ds(k * 2 * SUB, 2 * SUB)]], eb, gsem)
        cr = pltpu.async_copy(rel_hbm.at[ridx.at[pl.ds(k * SUB, SUB)]], rb, rsem)
        return (cp, cr)

    ebufs = (eb0, eb1)
    rbufs = (rb0, rb1)
    pending = [fire_phase(0, eb0, rb0), fire_phase(1, eb1, rb1)]

    def compute_phase(k, eb, rb):
        sout = psb if k < NPHASE // 2 else nsb
        sbase = (k % (NPHASE // 2)) * SUB

        def g_body(g, carry):
            hrows = g * L + lax.iota(jnp.int32, L)
            trows = SUB + hrows
            hbase = (ihb[pl.ds(k * SUB + g * L, L)] & 1) * DIM
            tbase = (itb[pl.ds(k * SUB + g * L, L)] & 1) * DIM
            rbase = (irb[pl.ds(k * SUB + g * L, L)] & 1) * DIM

            def j_body(j, acc):
                h2, t2, r2, hr, tr, ht = acc
                jv = jnp.full((L,), j, dtype=jnp.int32)
                hv = plsc.load_gather(eb, [hrows, hbase + jv])
                tv = plsc.load_gather(eb, [trows, tbase + jv])
                rv = plsc.load_gather(rb, [hrows, rbase + jv])
                return (h2 + hv * hv, t2 + tv * tv, r2 + rv * rv,
                        hr + hv * rv, tr + tv * rv, ht + hv * tv)

            zeros = jnp.zeros((L,), jnp.float32)
            h2, t2, r2, hr, tr, ht = lax.fori_loop(
                0, DIM, j_body, (zeros,) * 6, unroll=8)

            sh = jnp.minimum(1.0, _rsqrt_newton(h2))
            st = jnp.minimum(1.0, _rsqrt_newton(t2))
            dsq = (sh * sh * h2 + r2 + st * st * t2
                   + 2.0 * sh * hr - 2.0 * st * tr - 2.0 * (sh * st) * ht)
            dsq = jnp.maximum(dsq, 0.0)
            sout[pl.ds(sbase + g * L, L)] = dsq * _rsqrt_newton(dsq)
            return carry

        lax.fori_loop(0, NG, g_body, 0)

    for k in range(NPHASE):
        cp, cr = pending[k]
        cp.wait()
        cr.wait()
        compute_phase(k, ebufs[k % 2], rbufs[k % 2])
        nxt = k + 2
        if nxt < NPHASE:
            pending.append(fire_phase(nxt, ebufs[nxt % 2], rbufs[nxt % 2]))

    def l_body(g, acc):
        p = psb[pl.ds(g * L, L)]
        n = nsb[pl.ds(g * L, L)]
        return acc + jnp.maximum(0.0, p - n + MARGIN)

    lacc = lax.fori_loop(0, CHUNK // L, l_body, jnp.zeros((L,), jnp.float32))
    pvec[...] = jnp.full((L,), jnp.sum(lacc), jnp.float32)

    pltpu.sync_copy(psb, pos_out.at[pl.ds(pbase, CHUNK)])
    pltpu.sync_copy(nsb, neg_out.at[pl.ds(pbase, CHUNK)])
    pltpu.sync_copy(pvec, part_out.at[wid])


@jax.jit
def _transe_sc(bh, bt, br, ent2, rel2):
    mesh = plsc.VectorSubcoreMesh(
        core_axis_name="c", subcore_axis_name="s",
        num_cores=NC, num_subcores=NS)
    f = pl.kernel(
        _transe_body,
        out_type=(
            jax.ShapeDtypeStruct((HALF,), jnp.float32),
            jax.ShapeDtypeStruct((HALF,), jnp.float32),
            jax.ShapeDtypeStruct((NW, L), jnp.float32),
        ),
        mesh=mesh,
        compiler_params=pltpu.CompilerParams(
            needs_layout_passes=False, use_tc_tiling_on_sc=True),
        scratch_types=(
            [pltpu.VMEM((2 * CHUNK,), jnp.int32) for _ in range(3)]
            + [pltpu.VMEM((NPHASE * 2 * SUB,), jnp.int32)]
            + [pltpu.VMEM((2 * CHUNK,), jnp.int32)]
            + [pltpu.VMEM((2 * SUB, 2 * DIM), jnp.float32) for _ in range(2)]
            + [pltpu.VMEM((SUB, 2 * DIM), jnp.float32) for _ in range(2)]
            + [pltpu.VMEM((CHUNK,), jnp.float32) for _ in range(2)]
            + [pltpu.VMEM((L,), jnp.float32),
               pltpu.SemaphoreType.DMA, pltpu.SemaphoreType.DMA]
        ),
    )
    return f(bh, bt, br, ent2, rel2)


def kernel(batch_h, batch_t, batch_r, batch_y, ent_emb, rel_emb):
    del batch_y
    bh = batch_h.astype(jnp.int32)
    bt = batch_t.astype(jnp.int32)
    br = batch_r.astype(jnp.int32)
    ent2 = ent_emb.reshape(ent_emb.shape[0] // 2, 2 * DIM)
    rel2 = rel_emb.reshape(rel_emb.shape[0] // 2, 2 * DIM)
    pos_score, neg_score, partials = _transe_sc(bh, bt, br, ent2, rel2)
    loss = jnp.sum(partials[:, 0])
    return (loss, pos_score, neg_score)
